# baseline (speedup 1.0000x reference)
import functools

import jax
import jax.numpy as jnp
from jax import lax
from jax.experimental import pallas as pl
from jax.experimental.pallas import tpu as pltpu
from jax.experimental.pallas import tpu_sc as plsc

N = 10000
E = 320000
HDIM = 128
EDIM = 16
ZDIM = 32
NODE_NUM = 100
LAYERS = 3

NUM_SC = 2
NUM_TILES = 16
NW = NUM_SC * NUM_TILES
EPW = E // NW
K = 40
NCHUNK = EPW // K
NCHUNK_PAD = 264
SUPER = 24
NSUPER = NCHUNK_PAD // SUPER
ROWS_PER_TILE = 640
PADN = ROWS_PER_TILE * NUM_TILES

_F32 = jnp.float32


def _dense0_body(h_ref, wki_ref, wkj_ref, wr1_ref, br1_ref, wr2_ref, br2_ref,
                 a_ref, b_ref, r_ref):
    h = h_ref[...]
    a_ref[...] = jnp.dot(h, wki_ref[...], preferred_element_type=_F32)
    b_ref[...] = jnp.dot(h, wkj_ref[...], preferred_element_type=_F32)
    t = jnp.dot(h, wr1_ref[...], preferred_element_type=_F32) + br1_ref[...]
    r_ref[...] = jnp.dot(t, wr2_ref[...], preferred_element_type=_F32) + br2_ref[...]


def _denseL_body(rp_ref, agg_ref, wki_ref, wkj_ref, wr1_ref, br1_ref, wr2_ref,
                 br2_ref, a_ref, b_ref, r_ref):
    h = rp_ref[...] + agg_ref[0] + agg_ref[1]
    a_ref[...] = jnp.dot(h, wki_ref[...], preferred_element_type=_F32)
    b_ref[...] = jnp.dot(h, wkj_ref[...], preferred_element_type=_F32)
    t = jnp.dot(h, wr1_ref[...], preferred_element_type=_F32) + br1_ref[...]
    r_ref[...] = jnp.dot(t, wr2_ref[...], preferred_element_type=_F32) + br2_ref[...]


_BR = 1000

_W_SPEC = pl.BlockSpec((HDIM, HDIM), lambda i: (0, 0))
_BIAS_SPEC = pl.BlockSpec((1, HDIM), lambda i: (0, 0))
_ROW_SPEC = pl.BlockSpec((_BR, HDIM), lambda i: (i, 0))
_AGG_SPEC = pl.BlockSpec((NUM_SC, _BR, HDIM), lambda i: (0, i, 0))
_OUT3 = [jax.ShapeDtypeStruct((N, HDIM), _F32)] * 3


def _dense0(h, wki, wkj, wr1, br1, wr2, br2):
    return pl.pallas_call(
        _dense0_body,
        grid=(N // _BR,),
        in_specs=[_ROW_SPEC, _W_SPEC, _W_SPEC, _W_SPEC, _BIAS_SPEC, _W_SPEC,
                  _BIAS_SPEC],
        out_specs=[_ROW_SPEC, _ROW_SPEC, _ROW_SPEC],
        out_shape=_OUT3,
    )(h, wki, wkj, wr1, br1, wr2, br2)


def _denseL(r_prev, agg, wki, wkj, wr1, br1, wr2, br2):
    return pl.pallas_call(
        _denseL_body,
        grid=(N // _BR,),
        in_specs=[_ROW_SPEC, _AGG_SPEC, _W_SPEC, _W_SPEC, _W_SPEC, _BIAS_SPEC,
                  _W_SPEC, _BIAS_SPEC],
        out_specs=[_ROW_SPEC, _ROW_SPEC, _ROW_SPEC],
        out_shape=_OUT3,
    )(r_prev, agg, wki, wkj, wr1, br1, wr2, br2)


def _edgec_body(ea_ref, wke_ref, c_ref):
    c_ref[...] = jnp.dot(ea_ref[...], wke_ref[...], preferred_element_type=_F32)


_BE = 2000


def _edge_c(ea, wke):
    return pl.pallas_call(
        _edgec_body,
        grid=(E // _BE,),
        in_specs=[pl.BlockSpec((_BE, EDIM), lambda i: (i, 0)),
                  pl.BlockSpec((EDIM, HDIM), lambda i: (0, 0))],
        out_specs=pl.BlockSpec((_BE, HDIM), lambda i: (i, 0)),
        out_shape=jax.ShapeDtypeStruct((E, HDIM), _F32),
    )(ea, wke)


def _pool_body(rp_ref, agg_ref, w3_ref, b3_ref, w4_ref, b4_ref, mu_ref, lv_ref):
    h = rp_ref[...] + agg_ref[0] + agg_ref[1]
    hg = jnp.sum(h, axis=1)
    mu_ref[...] = jnp.dot(hg, w3_ref[...], preferred_element_type=_F32) + b3_ref[...]
    lv_ref[...] = jnp.dot(hg, w4_ref[...], preferred_element_type=_F32) + b4_ref[...]


def _pool(r_prev, agg, w3, b3, w4, b4):
    ngraph = N // NODE_NUM
    return pl.pallas_call(
        _pool_body,
        out_shape=[jax.ShapeDtypeStruct((ngraph, ZDIM), _F32)] * 2,
    )(r_prev.reshape(ngraph, NODE_NUM, HDIM),
      agg.reshape(NUM_SC, ngraph, NODE_NUM, HDIM), w3, b3, w4, b4)


def _edge_body(a_hbm, b_hbm, c_hbm, dst3_hbm, src3_hbm, out_hbm,
               dstv, srcv, arow, brow, crow, aggsh,
               sema, semb, semc, semsc0, semsc1, semsc2):
    c = lax.axis_index("c")
    s = lax.axis_index("s")
    wid = c * NUM_TILES + s
    scsems = (semsc0, semsc1, semsc2)

    def zfill(i, carry):
        for j in range(HDIM // 16):
            arow[0, i, pl.ds(j * 16, 16)] = jnp.zeros((16,), _F32)
        return carry
    lax.fori_loop(0, K, zfill, 0)

    def zcopy(i, carry):
        pltpu.sync_copy(arow.at[0],
                        aggsh.at[pl.ds(s * ROWS_PER_TILE + i * K, K)])
        return carry
    lax.fori_loop(0, ROWS_PER_TILE // K, zcopy, 0)
    plsc.subcore_barrier()

    def do_super(u, carry):
        pltpu.sync_copy(dst3_hbm.at[wid, pl.ds(u * SUPER, SUPER)], dstv)
        pltpu.sync_copy(src3_hbm.at[wid, pl.ds(u * SUPER, SUPER)], srcv)

        def cbase(i):
            gc = lax.min(u * SUPER + i, NCHUNK - 1)
            return wid * EPW + gc * K

        def fetch_ab(i, p):
            pltpu.async_copy(a_hbm.at[dstv.at[i]], arow.at[p], sema)
            pltpu.async_copy(b_hbm.at[srcv.at[i]], brow.at[p], semb)

        def fetch_c(i, p):
            pltpu.async_copy(c_hbm.at[pl.ds(cbase(i), K)], crow.at[p], semc)

        def wait_ab(p):
            pltpu.make_async_copy(a_hbm.at[dstv.at[0]], arow.at[p], sema).wait()
            pltpu.make_async_copy(b_hbm.at[srcv.at[0]], brow.at[p], semb).wait()

        def wait_c(p):
            pltpu.make_async_copy(c_hbm.at[pl.ds(0, K)], crow.at[p], semc).wait()

        def wait_scatter(p):
            pltpu.make_async_copy(arow.at[p], aggsh.at[dstv.at[0]],
                                  scsems[p]).wait()

        fetch_ab(0, 0)
        fetch_ab(1, 1)
        fetch_c(0, 0)

        def six(q6, carry6):
            for b in range(6):
                i = q6 * 6 + b
                p3 = b % 3
                p2 = b % 2
                wait_ab(p3)
                wait_c(p2)
                if b == 0:
                    @pl.when(q6 > 0)
                    def _():
                        wait_scatter(2)
                else:
                    wait_scatter((p3 + 2) % 3)
                fetch_ab(lax.min(i + 2, SUPER - 1), (p3 + 2) % 3)
                fetch_c(lax.min(i + 1, SUPER - 1), 1 - p2)

                gc = u * SUPER + i
                scale = jnp.where(gc < NCHUNK, _F32(1.0), _F32(0.0))

                def edge(e, ecarry, _p3=p3, _p2=p2):
                    for j in range(HDIM // 16):
                        sl = pl.ds(j * 16, 16)
                        t = (arow[_p3, e, sl] + brow[_p3, e, sl]
                             + crow[_p2, e, sl])
                        m = jnp.where(t >= 0.0, t, t * _F32(0.01))
                        arow[_p3, e, sl] = m * scale
                    return ecarry
                lax.fori_loop(0, K, edge, 0)

                pltpu.async_copy(arow.at[p3], aggsh.at[dstv.at[i]], scsems[p3],
                                 add=True)
            return carry6
        lax.fori_loop(0, SUPER // 6, six, 0)
        wait_ab(0)
        wait_ab(1)
        wait_c(0)
        wait_scatter(2)
        return carry
    lax.fori_loop(0, NSUPER, do_super, 0)

    plsc.subcore_barrier()
    pltpu.sync_copy(aggsh.at[pl.ds(s * ROWS_PER_TILE, ROWS_PER_TILE)],
                    out_hbm.at[c, pl.ds(s * ROWS_PER_TILE, ROWS_PER_TILE)])


_edge_kernel = functools.partial(
    pl.kernel,
    out_type=jax.ShapeDtypeStruct((NUM_SC, PADN, HDIM), _F32),
    mesh=plsc.VectorSubcoreMesh(core_axis_name="c", subcore_axis_name="s",
                                num_cores=NUM_SC, num_subcores=NUM_TILES),
    scratch_types=[
        pltpu.VMEM((SUPER, K), jnp.int32),
        pltpu.VMEM((SUPER, K), jnp.int32),
        pltpu.VMEM((3, K, HDIM), _F32),
        pltpu.VMEM((3, K, HDIM), _F32),
        pltpu.VMEM((2, K, HDIM), _F32),
        pltpu.VMEM_SHARED((PADN, HDIM), _F32),
        pltpu.SemaphoreType.DMA,
        pltpu.SemaphoreType.DMA,
        pltpu.SemaphoreType.DMA,
        pltpu.SemaphoreType.DMA,
        pltpu.SemaphoreType.DMA,
        pltpu.SemaphoreType.DMA,
    ],
)(_edge_body)


def kernel(x, edge_index, edge_attr, batch, Wr1, br1, Wr2, br2, Wk, W3, b3,
           W4, b4):
    del batch
    src = edge_index[0].astype(jnp.int32)
    dst = edge_index[1].astype(jnp.int32)
    pad = jnp.zeros((NW, (NCHUNK_PAD - NCHUNK) * K), jnp.int32)
    dst3 = jnp.concatenate([dst.reshape(NW, EPW), pad], axis=1).reshape(
        NW, NCHUNK_PAD, K)
    src3 = jnp.concatenate([src.reshape(NW, EPW), pad], axis=1).reshape(
        NW, NCHUNK_PAD, K)

    r_prev = None
    agg = None
    for l in range(LAYERS):
        wki = Wk[l, :HDIM, :]
        wkj = Wk[l, HDIM:2 * HDIM, :]
        wke = Wk[l, 2 * HDIM:, :]
        br1l = br1[l].reshape(1, HDIM)
        br2l = br2[l].reshape(1, HDIM)
        if l == 0:
            a, b, r = _dense0(x, wki, wkj, Wr1[l], br1l, Wr2[l], br2l)
        else:
            a, b, r = _denseL(r_prev, agg, wki, wkj, Wr1[l], br1l, Wr2[l], br2l)
        cmat = _edge_c(edge_attr, wke)
        agg = _edge_kernel(a, b, cmat, dst3, src3)
        r_prev = r

    mu, logvar = _pool(r_prev, agg[:, :N, :], W3, b3.reshape(1, ZDIM), W4,
                       b4.reshape(1, ZDIM))
    return (mu, logvar)


# speedup vs baseline: 1.8727x; 1.8727x over previous
import functools

import jax
import jax.numpy as jnp
from jax import lax
from jax.experimental import pallas as pl
from jax.experimental.pallas import tpu as pltpu
from jax.experimental.pallas import tpu_sc as plsc

N = 10000
E = 320000
HDIM = 128
EDIM = 16
ZDIM = 32
NODE_NUM = 100
LAYERS = 3

NUM_SC = 2
NUM_TILES = 16
NW = NUM_SC * NUM_TILES
EPW = E // NW
K = 40
NCHUNK = EPW // K
ROWS_PER_TILE = 640
PADN = ROWS_PER_TILE * NUM_TILES

_F32 = jnp.float32


def _dense0_body(h_ref, wki_ref, wkj_ref, wr1_ref, br1_ref, wr2_ref, br2_ref,
                 a_ref, b_ref, r_ref):
    h = h_ref[...]
    a_ref[...] = jnp.dot(h, wki_ref[...], preferred_element_type=_F32)
    b_ref[...] = jnp.dot(h, wkj_ref[...], preferred_element_type=_F32)
    t = jnp.dot(h, wr1_ref[...], preferred_element_type=_F32) + br1_ref[...]
    r_ref[...] = jnp.dot(t, wr2_ref[...], preferred_element_type=_F32) + br2_ref[...]


def _denseL_body(rp_ref, agg_ref, wki_ref, wkj_ref, wr1_ref, br1_ref, wr2_ref,
                 br2_ref, a_ref, b_ref, r_ref):
    h = rp_ref[...] + agg_ref[0] + agg_ref[1]
    a_ref[...] = jnp.dot(h, wki_ref[...], preferred_element_type=_F32)
    b_ref[...] = jnp.dot(h, wkj_ref[...], preferred_element_type=_F32)
    t = jnp.dot(h, wr1_ref[...], preferred_element_type=_F32) + br1_ref[...]
    r_ref[...] = jnp.dot(t, wr2_ref[...], preferred_element_type=_F32) + br2_ref[...]


_BR = 1000

_W_SPEC = pl.BlockSpec((HDIM, HDIM), lambda i: (0, 0))
_BIAS_SPEC = pl.BlockSpec((1, HDIM), lambda i: (0, 0))
_ROW_SPEC = pl.BlockSpec((_BR, HDIM), lambda i: (i, 0))
_AGG_SPEC = pl.BlockSpec((NUM_SC, _BR, HDIM), lambda i: (0, i, 0))
_OUT3 = [jax.ShapeDtypeStruct((N, HDIM), _F32)] * 3


def _dense0(h, wki, wkj, wr1, br1, wr2, br2):
    return pl.pallas_call(
        _dense0_body,
        grid=(N // _BR,),
        in_specs=[_ROW_SPEC, _W_SPEC, _W_SPEC, _W_SPEC, _BIAS_SPEC, _W_SPEC,
                  _BIAS_SPEC],
        out_specs=[_ROW_SPEC, _ROW_SPEC, _ROW_SPEC],
        out_shape=_OUT3,
    )(h, wki, wkj, wr1, br1, wr2, br2)


def _denseL(r_prev, agg, wki, wkj, wr1, br1, wr2, br2):
    return pl.pallas_call(
        _denseL_body,
        grid=(N // _BR,),
        in_specs=[_ROW_SPEC, _AGG_SPEC, _W_SPEC, _W_SPEC, _W_SPEC, _BIAS_SPEC,
                  _W_SPEC, _BIAS_SPEC],
        out_specs=[_ROW_SPEC, _ROW_SPEC, _ROW_SPEC],
        out_shape=_OUT3,
    )(r_prev, agg, wki, wkj, wr1, br1, wr2, br2)


def _edgec_body(ea_ref, wke_ref, c_ref):
    c_ref[...] = jnp.dot(ea_ref[...], wke_ref[...], preferred_element_type=_F32)


_BE = 2000


def _edge_c(ea, wke):
    return pl.pallas_call(
        _edgec_body,
        grid=(E // _BE,),
        in_specs=[pl.BlockSpec((_BE, EDIM), lambda i: (i, 0)),
                  pl.BlockSpec((EDIM, HDIM), lambda i: (0, 0))],
        out_specs=pl.BlockSpec((_BE, HDIM), lambda i: (i, 0)),
        out_shape=jax.ShapeDtypeStruct((E, HDIM), _F32),
    )(ea, wke)


def _pool_body(rp_ref, agg_ref, w3_ref, b3_ref, w4_ref, b4_ref, mu_ref, lv_ref):
    h = rp_ref[...] + agg_ref[0] + agg_ref[1]
    hg = jnp.sum(h, axis=1)
    mu_ref[...] = jnp.dot(hg, w3_ref[...], preferred_element_type=_F32) + b3_ref[...]
    lv_ref[...] = jnp.dot(hg, w4_ref[...], preferred_element_type=_F32) + b4_ref[...]


def _pool(r_prev, agg, w3, b3, w4, b4):
    ngraph = N // NODE_NUM
    return pl.pallas_call(
        _pool_body,
        out_shape=[jax.ShapeDtypeStruct((ngraph, ZDIM), _F32)] * 2,
    )(r_prev.reshape(ngraph, NODE_NUM, HDIM),
      agg.reshape(NUM_SC, ngraph, NODE_NUM, HDIM), w3, b3, w4, b4)


_ZROWS = 64


def _edge_body(a_hbm, b_hbm, c_hbm, dst_hbm, src_hbm, out_hbm,
               dstv, srcv, arow, brow, crow, zrow, aggsh,
               sema, semb, semc, semsc0, semsc1):
    c = lax.axis_index("c")
    s = lax.axis_index("s")
    wid = c * NUM_TILES + s
    scsems = (semsc0, semsc1)

    def zfill(i, carry):
        for j in range(HDIM // 16):
            zrow[i, pl.ds(j * 16, 16)] = jnp.zeros((16,), _F32)
        return carry
    lax.fori_loop(0, _ZROWS, zfill, 0)

    def zcopy(i, carry):
        pltpu.sync_copy(zrow, aggsh.at[pl.ds(s * ROWS_PER_TILE + i * _ZROWS, _ZROWS)])
        return carry
    lax.fori_loop(0, ROWS_PER_TILE // _ZROWS, zcopy, 0)
    plsc.subcore_barrier()

    ebase = wid * EPW

    def fetch(g, p):
        base = ebase + g * K
        pltpu.sync_copy(dst_hbm.at[pl.ds(base, K)], dstv.at[p])
        pltpu.sync_copy(src_hbm.at[pl.ds(base, K)], srcv.at[p])
        pltpu.async_copy(a_hbm.at[dstv.at[p]], arow.at[p], sema)
        pltpu.async_copy(b_hbm.at[srcv.at[p]], brow.at[p], semb)
        pltpu.async_copy(c_hbm.at[pl.ds(base, K)], crow.at[p], semc)

    def wait_fetch(p):
        pltpu.make_async_copy(a_hbm.at[dstv.at[p]], arow.at[p], sema).wait()
        pltpu.make_async_copy(b_hbm.at[srcv.at[p]], brow.at[p], semb).wait()
        pltpu.make_async_copy(c_hbm.at[pl.ds(0, K)], crow.at[p], semc).wait()

    def wait_scatter(p):
        pltpu.make_async_copy(arow.at[p], aggsh.at[dstv.at[p]],
                              scsems[p]).wait()

    fetch(0, 0)

    def chunk2(g2, carry):
        for b in range(2):
            g = g2 * 2 + b
            wait_fetch(b)
            if b == 0:
                @pl.when(g2 > 0)
                def _():
                    wait_scatter(1)
            else:
                wait_scatter(0)
            gnext = lax.min(g + 1, NCHUNK - 1)
            fetch(gnext, 1 - b)

            def edge(e, ecarry, _b=b):
                for j in range(HDIM // 16):
                    sl = pl.ds(j * 16, 16)
                    t = arow[_b, e, sl] + brow[_b, e, sl] + crow[_b, e, sl]
                    arow[_b, e, sl] = jnp.where(t >= 0.0, t, t * _F32(0.01))
                return ecarry
            lax.fori_loop(0, K, edge, 0)

            pltpu.async_copy(arow.at[b], aggsh.at[dstv.at[b]], scsems[b],
                             add=True)
        return carry
    lax.fori_loop(0, NCHUNK // 2, chunk2, 0)
    wait_fetch(0)
    wait_scatter(1)

    plsc.subcore_barrier()
    pltpu.sync_copy(aggsh.at[pl.ds(s * ROWS_PER_TILE, ROWS_PER_TILE)],
                    out_hbm.at[c, pl.ds(s * ROWS_PER_TILE, ROWS_PER_TILE)])


_edge_kernel = functools.partial(
    pl.kernel,
    out_type=jax.ShapeDtypeStruct((NUM_SC, PADN, HDIM), _F32),
    mesh=plsc.VectorSubcoreMesh(core_axis_name="c", subcore_axis_name="s",
                                num_cores=NUM_SC, num_subcores=NUM_TILES),
    scratch_types=[
        pltpu.VMEM((2, K), jnp.int32),
        pltpu.VMEM((2, K), jnp.int32),
        pltpu.VMEM((2, K, HDIM), _F32),
        pltpu.VMEM((2, K, HDIM), _F32),
        pltpu.VMEM((2, K, HDIM), _F32),
        pltpu.VMEM((_ZROWS, HDIM), _F32),
        pltpu.VMEM_SHARED((PADN, HDIM), _F32),
        pltpu.SemaphoreType.DMA,
        pltpu.SemaphoreType.DMA,
        pltpu.SemaphoreType.DMA,
        pltpu.SemaphoreType.DMA,
        pltpu.SemaphoreType.DMA,
    ],
)(_edge_body)


def kernel(x, edge_index, edge_attr, batch, Wr1, br1, Wr2, br2, Wk, W3, b3,
           W4, b4):
    del batch
    src = edge_index[0].astype(jnp.int32)
    dst = edge_index[1].astype(jnp.int32)

    r_prev = None
    agg = None
    for l in range(LAYERS):
        wki = Wk[l, :HDIM, :]
        wkj = Wk[l, HDIM:2 * HDIM, :]
        wke = Wk[l, 2 * HDIM:, :]
        br1l = br1[l].reshape(1, HDIM)
        br2l = br2[l].reshape(1, HDIM)
        if l == 0:
            a, b, r = _dense0(x, wki, wkj, Wr1[l], br1l, Wr2[l], br2l)
        else:
            a, b, r = _denseL(r_prev, agg, wki, wkj, Wr1[l], br1l, Wr2[l], br2l)
        cmat = _edge_c(edge_attr, wke)
        agg = _edge_kernel(a, b, cmat, dst, src)
        r_prev = r

    mu, logvar = _pool(r_prev, agg[:, :N, :], W3, b3.reshape(1, ZDIM), W4,
                       b4.reshape(1, ZDIM))
    return (mu, logvar)


# speedup vs baseline: 2.0864x; 1.1141x over previous
import functools

import jax
import jax.numpy as jnp
from jax import lax
from jax.experimental import pallas as pl
from jax.experimental.pallas import tpu as pltpu
from jax.experimental.pallas import tpu_sc as plsc

N = 10000
E = 320000
HDIM = 128
EDIM = 16
ZDIM = 32
NODE_NUM = 100
LAYERS = 3

NUM_SC = 2
NUM_TILES = 16
NW = NUM_SC * NUM_TILES
EPW = E // NW
K = 40
NCHUNK = EPW // K
ROWS_PER_TILE = 640
PADN = ROWS_PER_TILE * NUM_TILES

_F32 = jnp.float32


def _dense0_body(h_ref, wki_ref, wkj_ref, wr1_ref, br1_ref, wr2_ref, br2_ref,
                 a_ref, b_ref, r_ref):
    h = h_ref[...]
    a_ref[...] = jnp.dot(h, wki_ref[...], preferred_element_type=_F32)
    b_ref[...] = jnp.dot(h, wkj_ref[...], preferred_element_type=_F32)
    t = jnp.dot(h, wr1_ref[...], preferred_element_type=_F32) + br1_ref[...]
    r_ref[...] = jnp.dot(t, wr2_ref[...], preferred_element_type=_F32) + br2_ref[...]


def _denseL_body(rp_ref, agg_ref, wki_ref, wkj_ref, wr1_ref, br1_ref, wr2_ref,
                 br2_ref, a_ref, b_ref, r_ref):
    h = rp_ref[...] + agg_ref[0] + agg_ref[1]
    a_ref[...] = jnp.dot(h, wki_ref[...], preferred_element_type=_F32)
    b_ref[...] = jnp.dot(h, wkj_ref[...], preferred_element_type=_F32)
    t = jnp.dot(h, wr1_ref[...], preferred_element_type=_F32) + br1_ref[...]
    r_ref[...] = jnp.dot(t, wr2_ref[...], preferred_element_type=_F32) + br2_ref[...]


_BR = 1000

_W_SPEC = pl.BlockSpec((HDIM, HDIM), lambda i: (0, 0))
_BIAS_SPEC = pl.BlockSpec((1, HDIM), lambda i: (0, 0))
_ROW_SPEC = pl.BlockSpec((_BR, HDIM), lambda i: (i, 0))
_AGG_SPEC = pl.BlockSpec((NUM_SC, _BR, HDIM), lambda i: (0, i, 0))
_OUT3 = [jax.ShapeDtypeStruct((N, HDIM), _F32)] * 3


def _dense0(h, wki, wkj, wr1, br1, wr2, br2):
    return pl.pallas_call(
        _dense0_body,
        grid=(N // _BR,),
        in_specs=[_ROW_SPEC, _W_SPEC, _W_SPEC, _W_SPEC, _BIAS_SPEC, _W_SPEC,
                  _BIAS_SPEC],
        out_specs=[_ROW_SPEC, _ROW_SPEC, _ROW_SPEC],
        out_shape=_OUT3,
    )(h, wki, wkj, wr1, br1, wr2, br2)


def _denseL(r_prev, agg, wki, wkj, wr1, br1, wr2, br2):
    return pl.pallas_call(
        _denseL_body,
        grid=(N // _BR,),
        in_specs=[_ROW_SPEC, _AGG_SPEC, _W_SPEC, _W_SPEC, _W_SPEC, _BIAS_SPEC,
                  _W_SPEC, _BIAS_SPEC],
        out_specs=[_ROW_SPEC, _ROW_SPEC, _ROW_SPEC],
        out_shape=_OUT3,
    )(r_prev, agg, wki, wkj, wr1, br1, wr2, br2)


def _edgec_body(ea_ref, wke_ref, c_ref):
    c_ref[...] = jnp.dot(ea_ref[...], wke_ref[...], preferred_element_type=_F32)


_BE = 2000


def _edge_c(ea, wke):
    return pl.pallas_call(
        _edgec_body,
        grid=(E // _BE,),
        in_specs=[pl.BlockSpec((_BE, EDIM), lambda i: (i, 0)),
                  pl.BlockSpec((EDIM, HDIM), lambda i: (0, 0))],
        out_specs=pl.BlockSpec((_BE, HDIM), lambda i: (i, 0)),
        out_shape=jax.ShapeDtypeStruct((E, HDIM), _F32),
    )(ea, wke)


def _pool_body(rp_ref, agg_ref, w3_ref, b3_ref, w4_ref, b4_ref, mu_ref, lv_ref):
    h = rp_ref[...] + agg_ref[0] + agg_ref[1]
    hg = jnp.sum(h, axis=1)
    mu_ref[...] = jnp.dot(hg, w3_ref[...], preferred_element_type=_F32) + b3_ref[...]
    lv_ref[...] = jnp.dot(hg, w4_ref[...], preferred_element_type=_F32) + b4_ref[...]


def _pool(r_prev, agg, w3, b3, w4, b4):
    ngraph = N // NODE_NUM
    return pl.pallas_call(
        _pool_body,
        out_shape=[jax.ShapeDtypeStruct((ngraph, ZDIM), _F32)] * 2,
    )(r_prev.reshape(ngraph, NODE_NUM, HDIM),
      agg.reshape(NUM_SC, ngraph, NODE_NUM, HDIM), w3, b3, w4, b4)


_ZROWS = 64


def _edge_body(a_hbm, b_hbm, c_hbm, dsi_hbm, out_hbm,
               idxv, arow, brow, crow, zrow, aggsh,
               sema, semb, semc, semsc0, semsc1):
    c = lax.axis_index("c")
    s = lax.axis_index("s")
    wid = c * NUM_TILES + s
    scsems = (semsc0, semsc1)

    def zfill(i, carry):
        for j in range(HDIM // 16):
            zrow[i, pl.ds(j * 16, 16)] = jnp.zeros((16,), _F32)
        return carry
    lax.fori_loop(0, _ZROWS, zfill, 0)

    def zcopy(i, carry):
        pltpu.sync_copy(zrow, aggsh.at[pl.ds(s * ROWS_PER_TILE + i * _ZROWS, _ZROWS)])
        return carry
    lax.fori_loop(0, ROWS_PER_TILE // _ZROWS, zcopy, 0)
    plsc.subcore_barrier()

    ebase = wid * EPW
    rbase = wid * NCHUNK

    def fetch(g, p):
        pltpu.sync_copy(dsi_hbm.at[rbase + g], idxv.at[p])
        pltpu.async_copy(a_hbm.at[idxv.at[p, 0]], arow.at[p], sema)
        pltpu.async_copy(b_hbm.at[idxv.at[p, 1]], brow.at[p], semb)
        pltpu.async_copy(c_hbm.at[pl.ds(ebase + g * K, K)], crow.at[p], semc)

    def wait_fetch(p):
        pltpu.make_async_copy(a_hbm.at[idxv.at[p, 0]], arow.at[p], sema).wait()
        pltpu.make_async_copy(b_hbm.at[idxv.at[p, 1]], brow.at[p], semb).wait()
        pltpu.make_async_copy(c_hbm.at[pl.ds(0, K)], crow.at[p], semc).wait()

    def wait_scatter(p):
        pltpu.make_async_copy(arow.at[p], aggsh.at[idxv.at[p, 0]],
                              scsems[p]).wait()

    fetch(0, 0)

    def chunk2(g2, carry):
        for b in range(2):
            g = g2 * 2 + b
            wait_fetch(b)
            if b == 0:
                @pl.when(g2 > 0)
                def _():
                    wait_scatter(1)
            else:
                wait_scatter(0)
            gnext = lax.min(g + 1, NCHUNK - 1)
            fetch(gnext, 1 - b)

            def edge(e2, ecarry, _b=b):
                for dd in range(2):
                    e = e2 * 2 + dd
                    for j in range(HDIM // 16):
                        sl = pl.ds(j * 16, 16)
                        t = arow[_b, e, sl] + brow[_b, e, sl] + crow[_b, e, sl]
                        arow[_b, e, sl] = jnp.where(t >= 0.0, t, t * _F32(0.01))
                return ecarry
            lax.fori_loop(0, K // 2, edge, 0)

            pltpu.async_copy(arow.at[b], aggsh.at[idxv.at[b, 0]], scsems[b],
                             add=True)
        return carry
    lax.fori_loop(0, NCHUNK // 2, chunk2, 0)
    wait_fetch(0)
    wait_scatter(1)

    plsc.subcore_barrier()
    pltpu.sync_copy(aggsh.at[pl.ds(s * ROWS_PER_TILE, ROWS_PER_TILE)],
                    out_hbm.at[c, pl.ds(s * ROWS_PER_TILE, ROWS_PER_TILE)])


_edge_kernel = functools.partial(
    pl.kernel,
    out_type=jax.ShapeDtypeStruct((NUM_SC, PADN, HDIM), _F32),
    mesh=plsc.VectorSubcoreMesh(core_axis_name="c", subcore_axis_name="s",
                                num_cores=NUM_SC, num_subcores=NUM_TILES),
    scratch_types=[
        pltpu.VMEM((2, 2, K), jnp.int32),
        pltpu.VMEM((2, K, HDIM), _F32),
        pltpu.VMEM((2, K, HDIM), _F32),
        pltpu.VMEM((2, K, HDIM), _F32),
        pltpu.VMEM((_ZROWS, HDIM), _F32),
        pltpu.VMEM_SHARED((PADN, HDIM), _F32),
        pltpu.SemaphoreType.DMA,
        pltpu.SemaphoreType.DMA,
        pltpu.SemaphoreType.DMA,
        pltpu.SemaphoreType.DMA,
        pltpu.SemaphoreType.DMA,
    ],
)(_edge_body)


def kernel(x, edge_index, edge_attr, batch, Wr1, br1, Wr2, br2, Wk, W3, b3,
           W4, b4):
    del batch
    src = edge_index[0].astype(jnp.int32)
    dst = edge_index[1].astype(jnp.int32)
    dsi = jnp.stack([dst.reshape(NW * NCHUNK, K),
                     src.reshape(NW * NCHUNK, K)], axis=1)

    r_prev = None
    agg = None
    for l in range(LAYERS):
        wki = Wk[l, :HDIM, :]
        wkj = Wk[l, HDIM:2 * HDIM, :]
        wke = Wk[l, 2 * HDIM:, :]
        br1l = br1[l].reshape(1, HDIM)
        br2l = br2[l].reshape(1, HDIM)
        if l == 0:
            a, b, r = _dense0(x, wki, wkj, Wr1[l], br1l, Wr2[l], br2l)
        else:
            a, b, r = _denseL(r_prev, agg, wki, wkj, Wr1[l], br1l, Wr2[l], br2l)
        cmat = _edge_c(edge_attr, wke)
        agg = _edge_kernel(a, b, cmat, dsi)
        r_prev = r

    mu, logvar = _pool(r_prev, agg[:, :N, :], W3, b3.reshape(1, ZDIM), W4,
                       b4.reshape(1, ZDIM))
    return (mu, logvar)


# speedup vs baseline: 2.0877x; 1.0007x over previous
import functools

import jax
import jax.numpy as jnp
from jax import lax
from jax.experimental import pallas as pl
from jax.experimental.pallas import tpu as pltpu
from jax.experimental.pallas import tpu_sc as plsc

N = 10000
E = 320000
HDIM = 128
EDIM = 16
ZDIM = 32
NODE_NUM = 100
LAYERS = 3

NUM_SC = 2
NUM_TILES = 16
NW = NUM_SC * NUM_TILES
EPW = E // NW
K = 40
NCHUNK = EPW // K
ROWS_PER_TILE = 640
PADN = ROWS_PER_TILE * NUM_TILES

_F32 = jnp.float32


def _dense0_body(h_ref, wki_ref, wkj_ref, wr1_ref, br1_ref, wr2_ref, br2_ref,
                 a_ref, b_ref, r_ref):
    h = h_ref[...]
    a_ref[...] = jnp.dot(h, wki_ref[...], preferred_element_type=_F32)
    b_ref[...] = jnp.dot(h, wkj_ref[...], preferred_element_type=_F32)
    t = jnp.dot(h, wr1_ref[...], preferred_element_type=_F32) + br1_ref[...]
    r_ref[...] = jnp.dot(t, wr2_ref[...], preferred_element_type=_F32) + br2_ref[...]


def _denseL_body(rp_ref, agg_ref, wki_ref, wkj_ref, wr1_ref, br1_ref, wr2_ref,
                 br2_ref, a_ref, b_ref, r_ref):
    h = rp_ref[...] + agg_ref[0] + agg_ref[1]
    a_ref[...] = jnp.dot(h, wki_ref[...], preferred_element_type=_F32)
    b_ref[...] = jnp.dot(h, wkj_ref[...], preferred_element_type=_F32)
    t = jnp.dot(h, wr1_ref[...], preferred_element_type=_F32) + br1_ref[...]
    r_ref[...] = jnp.dot(t, wr2_ref[...], preferred_element_type=_F32) + br2_ref[...]


_BR = 1000

_W_SPEC = pl.BlockSpec((HDIM, HDIM), lambda i: (0, 0))
_BIAS_SPEC = pl.BlockSpec((1, HDIM), lambda i: (0, 0))
_ROW_SPEC = pl.BlockSpec((_BR, HDIM), lambda i: (i, 0))
_AGG_SPEC = pl.BlockSpec((NUM_SC, _BR, HDIM), lambda i: (0, i, 0))
_OUT3 = [jax.ShapeDtypeStruct((N, HDIM), _F32)] * 3


def _dense0(h, wki, wkj, wr1, br1, wr2, br2):
    return pl.pallas_call(
        _dense0_body,
        grid=(N // _BR,),
        in_specs=[_ROW_SPEC, _W_SPEC, _W_SPEC, _W_SPEC, _BIAS_SPEC, _W_SPEC,
                  _BIAS_SPEC],
        out_specs=[_ROW_SPEC, _ROW_SPEC, _ROW_SPEC],
        out_shape=_OUT3,
    )(h, wki, wkj, wr1, br1, wr2, br2)


def _denseL(r_prev, agg, wki, wkj, wr1, br1, wr2, br2):
    return pl.pallas_call(
        _denseL_body,
        grid=(N // _BR,),
        in_specs=[_ROW_SPEC, _AGG_SPEC, _W_SPEC, _W_SPEC, _W_SPEC, _BIAS_SPEC,
                  _W_SPEC, _BIAS_SPEC],
        out_specs=[_ROW_SPEC, _ROW_SPEC, _ROW_SPEC],
        out_shape=_OUT3,
    )(r_prev, agg, wki, wkj, wr1, br1, wr2, br2)


def _edgec_body(ea_ref, wke_ref, c_ref):
    c_ref[...] = jnp.dot(ea_ref[...], wke_ref[...], preferred_element_type=_F32)


_BE = 2000


def _edge_c(ea, wke):
    return pl.pallas_call(
        _edgec_body,
        grid=(E // _BE,),
        in_specs=[pl.BlockSpec((_BE, EDIM), lambda i: (i, 0)),
                  pl.BlockSpec((EDIM, HDIM), lambda i: (0, 0))],
        out_specs=pl.BlockSpec((_BE, HDIM), lambda i: (i, 0)),
        out_shape=jax.ShapeDtypeStruct((E, HDIM), _F32),
    )(ea, wke)


def _pool_body(rp_ref, agg_ref, w3_ref, b3_ref, w4_ref, b4_ref, mu_ref, lv_ref):
    h = rp_ref[...] + agg_ref[0] + agg_ref[1]
    hg = jnp.sum(h, axis=1)
    mu_ref[...] = jnp.dot(hg, w3_ref[...], preferred_element_type=_F32) + b3_ref[...]
    lv_ref[...] = jnp.dot(hg, w4_ref[...], preferred_element_type=_F32) + b4_ref[...]


def _pool(r_prev, agg, w3, b3, w4, b4):
    ngraph = N // NODE_NUM
    return pl.pallas_call(
        _pool_body,
        out_shape=[jax.ShapeDtypeStruct((ngraph, ZDIM), _F32)] * 2,
    )(r_prev.reshape(ngraph, NODE_NUM, HDIM),
      agg.reshape(NUM_SC, ngraph, NODE_NUM, HDIM), w3, b3, w4, b4)


_ZROWS = 64


def _edge_body(a_hbm, b_hbm, c_hbm, dsi_hbm, out_hbm,
               idxv, arow, brow, crow, zrow, aggsh,
               sema, semb, semc, semsc0, semsc1):
    c = lax.axis_index("c")
    s = lax.axis_index("s")
    wid = c * NUM_TILES + s
    scsems = (semsc0, semsc1)

    def zfill(i, carry):
        for j in range(HDIM // 16):
            zrow[i, pl.ds(j * 16, 16)] = jnp.zeros((16,), _F32)
        return carry
    lax.fori_loop(0, _ZROWS, zfill, 0)

    def zcopy(i, carry):
        pltpu.sync_copy(zrow, aggsh.at[pl.ds(s * ROWS_PER_TILE + i * _ZROWS, _ZROWS)])
        return carry
    lax.fori_loop(0, ROWS_PER_TILE // _ZROWS, zcopy, 0)
    plsc.subcore_barrier()

    ebase = wid * EPW
    rbase = wid * NCHUNK

    def fetch(g, p):
        pltpu.sync_copy(dsi_hbm.at[rbase + g], idxv.at[p])
        pltpu.async_copy(a_hbm.at[idxv.at[p, 0]], arow.at[p], sema)
        pltpu.async_copy(b_hbm.at[idxv.at[p, 1]], brow.at[p], semb)
        pltpu.async_copy(c_hbm.at[pl.ds(ebase + g * K, K)], crow.at[p], semc)

    def wait_fetch(p):
        pltpu.make_async_copy(a_hbm.at[idxv.at[p, 0]], arow.at[p], sema).wait()
        pltpu.make_async_copy(b_hbm.at[idxv.at[p, 1]], brow.at[p], semb).wait()
        pltpu.make_async_copy(c_hbm.at[pl.ds(0, K)], crow.at[p], semc).wait()

    def wait_scatter(p):
        pltpu.make_async_copy(arow.at[p], aggsh.at[idxv.at[p, 0]],
                              scsems[p]).wait()

    fetch(0, 0)

    def chunk2(g2, carry):
        for b in range(2):
            g = g2 * 2 + b
            wait_fetch(b)
            if b == 0:
                @pl.when(g2 > 0)
                def _():
                    wait_scatter(1)
            else:
                wait_scatter(0)
            gnext = lax.min(g + 1, NCHUNK - 1)
            fetch(gnext, 1 - b)

            def edge(e4, ecarry, _b=b):
                for dd in range(4):
                    e = e4 * 4 + dd
                    for j in range(HDIM // 16):
                        sl = pl.ds(j * 16, 16)
                        t = arow[_b, e, sl] + brow[_b, e, sl] + crow[_b, e, sl]
                        arow[_b, e, sl] = jnp.where(t >= 0.0, t, t * _F32(0.01))
                return ecarry
            lax.fori_loop(0, K // 4, edge, 0)

            pltpu.async_copy(arow.at[b], aggsh.at[idxv.at[b, 0]], scsems[b],
                             add=True)
        return carry
    lax.fori_loop(0, NCHUNK // 2, chunk2, 0)
    wait_fetch(0)
    wait_scatter(1)

    plsc.subcore_barrier()
    pltpu.sync_copy(aggsh.at[pl.ds(s * ROWS_PER_TILE, ROWS_PER_TILE)],
                    out_hbm.at[c, pl.ds(s * ROWS_PER_TILE, ROWS_PER_TILE)])


_edge_kernel = functools.partial(
    pl.kernel,
    out_type=jax.ShapeDtypeStruct((NUM_SC, PADN, HDIM), _F32),
    mesh=plsc.VectorSubcoreMesh(core_axis_name="c", subcore_axis_name="s",
                                num_cores=NUM_SC, num_subcores=NUM_TILES),
    scratch_types=[
        pltpu.VMEM((2, 2, K), jnp.int32),
        pltpu.VMEM((2, K, HDIM), _F32),
        pltpu.VMEM((2, K, HDIM), _F32),
        pltpu.VMEM((2, K, HDIM), _F32),
        pltpu.VMEM((_ZROWS, HDIM), _F32),
        pltpu.VMEM_SHARED((PADN, HDIM), _F32),
        pltpu.SemaphoreType.DMA,
        pltpu.SemaphoreType.DMA,
        pltpu.SemaphoreType.DMA,
        pltpu.SemaphoreType.DMA,
        pltpu.SemaphoreType.DMA,
    ],
)(_edge_body)


def kernel(x, edge_index, edge_attr, batch, Wr1, br1, Wr2, br2, Wk, W3, b3,
           W4, b4):
    del batch
    src = edge_index[0].astype(jnp.int32)
    dst = edge_index[1].astype(jnp.int32)
    dsi = jnp.stack([dst.reshape(NW * NCHUNK, K),
                     src.reshape(NW * NCHUNK, K)], axis=1)

    r_prev = None
    agg = None
    for l in range(LAYERS):
        wki = Wk[l, :HDIM, :]
        wkj = Wk[l, HDIM:2 * HDIM, :]
        wke = Wk[l, 2 * HDIM:, :]
        br1l = br1[l].reshape(1, HDIM)
        br2l = br2[l].reshape(1, HDIM)
        if l == 0:
            a, b, r = _dense0(x, wki, wkj, Wr1[l], br1l, Wr2[l], br2l)
        else:
            a, b, r = _denseL(r_prev, agg, wki, wkj, Wr1[l], br1l, Wr2[l], br2l)
        cmat = _edge_c(edge_attr, wke)
        agg = _edge_kernel(a, b, cmat, dsi)
        r_prev = r

    mu, logvar = _pool(r_prev, agg[:, :N, :], W3, b3.reshape(1, ZDIM), W4,
                       b4.reshape(1, ZDIM))
    return (mu, logvar)
